# baseline (speedup 1.0000x reference)
import jax
import jax.numpy as jnp
from jax.experimental import pallas as pl

TAU_PRE = 20.0
TAU_POST = 20.0
A_PRE = 0.01
A_POST = 0.01
DT = 1.0


def _stdp_body(w_ref, pre_ref, post_ref, out_ref):
    T = pre_ref.shape[1]
    dtype = w_ref.dtype
    t1 = jax.lax.broadcasted_iota(jnp.int32, (T, T), 0).astype(dtype)
    t2 = jax.lax.broadcasted_iota(jnp.int32, (T, T), 1).astype(dtype)
    diff = (t2 - t1) * DT
    K = jnp.where(
        diff > 0,
        A_POST * jnp.exp(-diff / TAU_POST),
        jnp.where(diff < 0, -A_PRE * jnp.exp(diff / TAU_PRE), jnp.zeros_like(diff)),
    )
    B = pre_ref.shape[0]
    N = pre_ref.shape[2]
    acc = jnp.zeros((N, post_ref.shape[2]), dtype=jnp.float32)
    for b in range(B):
        pre_b = (pre_ref[b] != 0).astype(dtype)
        post_b = (post_ref[b] != 0).astype(dtype)
        m_b = jax.lax.dot_general(
            K, post_b,
            dimension_numbers=(((1,), (0,)), ((), ())),
            preferred_element_type=jnp.float32,
        )
        acc = acc + jax.lax.dot_general(
            pre_b, m_b,
            dimension_numbers=(((0,), (0,)), ((), ())),
            preferred_element_type=jnp.float32,
        )
    out_ref[...] = w_ref[...] + acc.astype(dtype)


def kernel(weights, pre_spikes, post_spikes):
    return pl.pallas_call(
        _stdp_body,
        out_shape=jax.ShapeDtypeStruct(weights.shape, weights.dtype),
    )(weights, pre_spikes, post_spikes)
